# baseline (speedup 1.0000x reference)
import functools

import jax
import jax.numpy as jnp
from jax import lax
from jax.experimental import pallas as pl
from jax.experimental.pallas import tpu as pltpu
from jax.experimental.pallas import tpu_sc as plsc

_NC, _NS, _L = 2, 16, 16
_NW = _NC * _NS
_D = 7
_N = 16384 * 200
_PER_W = _N // _NW
_CHUNK = 6400
_NCHUNK = _PER_W // _CHUNK
_G = _CHUNK // _L
_W = _L * _D


def _vreg_gather(vec, idx):
    return lax.gather(
        vec,
        idx[:, None],
        lax.GatherDimensionNumbers(
            offset_dims=(), collapsed_slice_dims=(0,), start_index_map=(0,)),
        slice_sizes=(1,),
        mode=lax.GatherScatterMode.PROMISE_IN_BOUNDS,
    )


def _sc_body(idx_hbm, tbl_hbm, oj_hbm, out_hbm, idx_buf, out_buf, tbl_v, oj_v):
    wid = lax.axis_index("s") * _NC + lax.axis_index("c")
    base = wid * _PER_W

    pltpu.sync_copy(tbl_hbm, tbl_v)
    pltpu.sync_copy(oj_hbm, oj_v)
    t0p = [tbl_v[q, :] for q in range(_D)]
    t1p = [tbl_v[_D + q, :] for q in range(_D)]
    ojp = [oj_v[q, :] for q in range(_D)]

    for c in range(_NCHUNK):
        cbase = base + c * _CHUNK
        pltpu.sync_copy(idx_hbm.at[pl.ds(cbase, _CHUNK)], idx_buf)

        def grp(p, carry):
            for h in range(2):
                e = idx_buf[pl.ds((2 * p + h) * _L, _L)]
                for q in range(_D):
                    eq = _vreg_gather(e, ojp[q])
                    v = jnp.where(eq != 0, t1p[q], t0p[q])
                    out_buf[p, pl.ds(h * _W + q * _L, _L)] = v
            return carry

        lax.fori_loop(0, _G // 2, grp, 0)
        pltpu.sync_copy(
            out_buf,
            out_hbm.at[pl.ds(pl.multiple_of(cbase // (2 * _L), 8), _G // 2)])


_lookup = functools.partial(
    pl.kernel,
    out_type=jax.ShapeDtypeStruct((_N * _D // (2 * _W), 2 * _W), jnp.float32),
    mesh=plsc.VectorSubcoreMesh(core_axis_name="c", subcore_axis_name="s"),
    scratch_types=[
        pltpu.VMEM((_CHUNK,), jnp.int32),
        pltpu.VMEM((_G // 2, 2 * _W), jnp.float32),
        pltpu.VMEM((2 * _D, _L), jnp.float32),
        pltpu.VMEM((_D, _L), jnp.int32),
    ],
)(_sc_body)


def kernel(indices, table):
    idx_flat = indices.reshape(-1).astype(jnp.int32)
    kpat = (jnp.arange(_D * _L, dtype=jnp.int32) % _D).reshape(_D, _L)
    tbl_b = jnp.concatenate([table[0][kpat], table[1][kpat]], axis=0)
    oj = (jnp.arange(_D * _L, dtype=jnp.int32) // _D).reshape(_D, _L)
    out2 = _lookup(idx_flat, tbl_b, oj)
    return out2.reshape(indices.shape[0], indices.shape[1], _D)


# speedup vs baseline: 1.5056x; 1.5056x over previous
import functools

import jax
import jax.numpy as jnp
from jax import lax
from jax.experimental import pallas as pl
from jax.experimental.pallas import tpu as pltpu
from jax.experimental.pallas import tpu_sc as plsc

_NC, _NS, _L = 2, 16, 16
_NW = _NC * _NS
_D = 7
_N = 16384 * 200
_PER_W = _N // _NW
_CHUNK = 6400
_NCHUNK = _PER_W // _CHUNK
_G = _CHUNK // _L
_OUT_CHUNK = _CHUNK * _D
_UW = 2 * 200 * _D
_GPU = 400 // _L
_UPC = _CHUNK // 400


def _vreg_gather(vec, idx):
    return lax.gather(
        vec,
        idx[:, None],
        lax.GatherDimensionNumbers(
            offset_dims=(), collapsed_slice_dims=(0,), start_index_map=(0,)),
        slice_sizes=(1,),
        mode=lax.GatherScatterMode.PROMISE_IN_BOUNDS,
    )


def _sc_body(idx_hbm, tbl_hbm, oj_hbm, out_hbm, idx_buf, out_buf, tbl_v, oj_v):
    wid = lax.axis_index("s") * _NC + lax.axis_index("c")
    base = wid * _PER_W

    pltpu.sync_copy(tbl_hbm, tbl_v)
    pltpu.sync_copy(oj_hbm, oj_v)
    t0p = [tbl_v[q, :] for q in range(_D)]
    t1p = [tbl_v[_D + q, :] for q in range(_D)]
    ojp = [oj_v[q, :] for q in range(_D)]

    for c in range(_NCHUNK):
        cbase = base + c * _CHUNK
        pltpu.sync_copy(idx_hbm.at[pl.ds(cbase, _CHUNK)], idx_buf)

        def unit(u, carry):
            def grp(g, carry2):
                e = idx_buf[pl.ds((u * _GPU + g) * _L, _L)]
                ob = g * (_L * _D)
                for q in range(_D):
                    eq = _vreg_gather(e, ojp[q])
                    v = jnp.where(eq != 0, t1p[q], t0p[q])
                    out_buf[u, pl.ds(ob + q * _L, _L)] = v
                return carry2
            return lax.fori_loop(0, _GPU, grp, carry)

        lax.fori_loop(0, _UPC, unit, 0)
        pltpu.sync_copy(
            out_buf,
            out_hbm.at[pl.ds(pl.multiple_of(cbase // 400, 8), _UPC)])


_lookup = functools.partial(
    pl.kernel,
    out_type=jax.ShapeDtypeStruct((16384 // 2, _UW), jnp.float32),
    mesh=plsc.VectorSubcoreMesh(core_axis_name="c", subcore_axis_name="s"),
    scratch_types=[
        pltpu.VMEM((_CHUNK,), jnp.int32),
        pltpu.VMEM((_UPC, _UW), jnp.float32),
        pltpu.VMEM((2 * _D, _L), jnp.float32),
        pltpu.VMEM((_D, _L), jnp.int32),
    ],
)(_sc_body)


def kernel(indices, table):
    idx_flat = indices.reshape(-1).astype(jnp.int32)
    kpat = (jnp.arange(_D * _L, dtype=jnp.int32) % _D).reshape(_D, _L)
    tbl_b = jnp.concatenate([table[0][kpat], table[1][kpat]], axis=0)
    oj = (jnp.arange(_D * _L, dtype=jnp.int32) // _D).reshape(_D, _L)
    out2 = _lookup(idx_flat, tbl_b, oj)
    return out2.reshape(indices.shape[0], indices.shape[1], _D)
